# baseline (speedup 1.0000x reference)
import jax
import jax.numpy as jnp
from jax.experimental import pallas as pl
from jax.experimental.pallas import tpu as pltpu

_B = 128
_V = 100000
_W = 8192
_NB = (_V + _W - 1) // _W


def _loss_kernel(x_ref, t_ref, o_ref, rmax, tsum):
    j = pl.program_id(0)

    @pl.when(j == 0)
    def _init():
        rmax[...] = jnp.full((_B, 1), -jnp.inf, jnp.float32)
        tsum[...] = jnp.zeros((_B, 1), jnp.float32)

    x = x_ref[...]
    cols = j * _W + jax.lax.broadcasted_iota(jnp.int32, (_B, _W), 1)
    t = t_ref[...]
    is_t = cols == t
    valid = cols < _V
    elem = jnp.where(valid, jnp.where(is_t, 0.0, x), -jnp.inf)
    rmax[...] = jnp.maximum(rmax[...], jnp.max(elem, axis=1, keepdims=True))
    tsum[...] += jnp.sum(jnp.where(is_t, x, 0.0), axis=1, keepdims=True)

    @pl.when(j == _NB - 1)
    def _fini():
        o_ref[0, 0] = jnp.sum(tsum[...] - rmax[...]) / _B


@jax.jit
def _run(output, t):
    return pl.pallas_call(
        _loss_kernel,
        grid=(_NB,),
        in_specs=[
            pl.BlockSpec((_B, _W), lambda j: (0, j)),
            pl.BlockSpec((_B, 1), lambda j: (0, 0)),
        ],
        out_specs=pl.BlockSpec(memory_space=pltpu.SMEM),
        out_shape=jax.ShapeDtypeStruct((1, 1), jnp.float32),
        scratch_shapes=[
            pltpu.VMEM((_B, 1), jnp.float32),
            pltpu.VMEM((_B, 1), jnp.float32),
        ],
    )(output, t)


def kernel(output, targetC):
    t = targetC.astype(jnp.int32).reshape(_B, 1)
    return _run(output, t)[0, 0]
